# baseline (speedup 1.0000x reference)
import functools

import jax
import jax.numpy as jnp
import numpy as np
from jax import lax
from jax.experimental import pallas as pl
from jax.experimental.pallas import tpu as pltpu
from jax.experimental.pallas import tpu_sc as plsc

N_ROWS = 1048576
N_CLS = 64
N_BINS = 15
BLK = 32768
GRID = N_ROWS // BLK
N_CHUNKS = 1
CGRID = GRID // N_CHUNKS
CROWS = N_ROWS // N_CHUNKS
NW = 32
PER_W = CROWS // NW

_BOUNDS = [float(v) for v in np.linspace(0.0, 1.0, N_BINS + 1)[1:N_BINS].astype(np.float32)]


SUB = BLK // 128


def _tc_body(xt_ref, lab_ref, word_ref):
    xt = xt_ref[...]
    m = jnp.max(xt, axis=0, keepdims=True)
    e = jnp.exp(xt)
    ones_w = jnp.full((1, N_CLS), 1.0, jnp.float32)
    s = jax.lax.dot_general(ones_w, e, (((1,), (0,)), ((), ())),
                            preferred_element_type=jnp.float32)
    eqm = jnp.where(xt == m, 1.0, 0.0)
    iota_w = lax.broadcasted_iota(jnp.int32, (1, N_CLS), 1).astype(jnp.float32)
    pred = jax.lax.dot_general(iota_w, eqm, (((1,), (0,)), ((), ())),
                               preferred_element_type=jnp.float32)

    conf = jnp.exp(m.reshape(SUB, 128)) / s.reshape(SUB, 128)
    pred2 = pred.reshape(SUB, 128)
    acci = (pred2 == lab_ref[0].astype(jnp.float32)).astype(jnp.int32)
    b = (conf > _BOUNDS[0]).astype(jnp.int32)
    for bk in _BOUNDS[1:]:
        b += (conf > bk).astype(jnp.int32)

    conf_q = (conf * 2097151.0).astype(jnp.int32)
    word_ref[0] = b * 67108864 + acci * 2097152 + conf_q


def _tc_stage(logits_t, labels3, chunk):
    word_sd = jax.ShapeDtypeStruct((CGRID, SUB, 128), jnp.int32)
    base = chunk * CGRID
    row_spec = pl.BlockSpec((1, SUB, 128), lambda i: (base + i, 0, 0))
    out_spec = pl.BlockSpec((1, SUB, 128), lambda i: (i, 0, 0))
    word = pl.pallas_call(
        _tc_body,
        grid=(CGRID,),
        in_specs=[
            pl.BlockSpec((N_CLS, BLK), lambda i: (0, base + i)),
            row_spec,
        ],
        out_specs=out_spec,
        out_shape=word_sd,
    )(logits_t, labels3)
    return word.reshape(CROWS)


def _sc_body(word_hbm, cout_hbm, zout_hbm,
             word_v, csum_a, z_a, csum_b, z_b, cf_h):
    wid = lax.axis_index("s") * 2 + lax.axis_index("c")
    base = wid * PER_W
    pltpu.sync_copy(word_hbm.at[pl.ds(base, PER_W)], word_v)

    zeros16i = jnp.zeros((16,), jnp.int32)
    for r in range(16):
        sl = pl.ds(r * 16, 16)
        csum_a[sl] = zeros16i
        z_a[sl] = zeros16i
        csum_b[sl] = zeros16i
        z_b[sl] = zeros16i

    lane = lax.broadcasted_iota(jnp.int32, (16,), 0)

    def step(i, carry):
        off = i * 32
        w0 = word_v[pl.ds(off, 16)]
        w1 = word_v[pl.ds(off + 16, 16)]
        idx0 = lax.shift_right_logical(w0, 22) + lane
        idx1 = lax.shift_right_logical(w1, 22) + lane
        plsc.addupdate_scatter(csum_a, [idx0], lax.bitwise_and(w0, 2097151))
        plsc.addupdate_scatter(z_a, [idx0],
                               lax.bitwise_or(lax.shift_right_logical(w0, 21),
                                              32768))
        plsc.addupdate_scatter(csum_b, [idx1], lax.bitwise_and(w1, 2097151))
        plsc.addupdate_scatter(z_b, [idx1],
                               lax.bitwise_or(lax.shift_right_logical(w1, 21),
                                              32768))
        return carry

    lax.fori_loop(0, PER_W // 32, step, 0, unroll=4)

    for r in range(16):
        sl = pl.ds(r * 16, 16)
        cf_h[sl] = (csum_a[sl].astype(jnp.float32) +
                    csum_b[sl].astype(jnp.float32))
        z_a[sl] = z_a[sl] + z_b[sl]

    obase = wid * 256
    pltpu.sync_copy(cf_h, cout_hbm.at[pl.ds(obase, 256)])
    pltpu.sync_copy(z_a, zout_hbm.at[pl.ds(obase, 256)])


def _sc_stage(word):
    mesh = plsc.VectorSubcoreMesh(core_axis_name="c", subcore_axis_name="s")
    kern = pl.kernel(
        _sc_body,
        out_type=(jax.ShapeDtypeStruct((NW * 256,), jnp.float32),
                  jax.ShapeDtypeStruct((NW * 256,), jnp.int32)),
        mesh=mesh,
        compiler_params=pltpu.CompilerParams(needs_layout_passes=False),
        scratch_types=[
            pltpu.VMEM((PER_W,), jnp.int32),
            pltpu.VMEM((256,), jnp.int32),
            pltpu.VMEM((256,), jnp.int32),
            pltpu.VMEM((256,), jnp.int32),
            pltpu.VMEM((256,), jnp.int32),
            pltpu.VMEM((256,), jnp.float32),
        ],
    )
    return kern(word)


def kernel(logits, labels):
    labels3 = labels.astype(jnp.int32).reshape(GRID, SUB, 128)
    logits_t = logits.T
    csum_f = jnp.zeros((NW * 256,), jnp.float32)
    za = jnp.zeros((NW * 256,), jnp.int32)
    for h in range(N_CHUNKS):
        word = _tc_stage(logits_t, labels3, h)
        csum_p, z_p = _sc_stage(word)
        csum_f = csum_f + csum_p
        za = za + z_p
    za3 = za.reshape(NW, 16, 16)
    denom = (32768 + 32 * jnp.arange(16, dtype=jnp.int32)).reshape(1, 16, 1)
    count_c = za3 // denom
    acc_c = za3 - count_c * denom
    count = jnp.sum(count_c, axis=(0, 2)).astype(jnp.float32)[:N_BINS]
    asum = jnp.sum(acc_c, axis=(0, 2)).astype(jnp.float32)[:N_BINS]
    csum3 = jnp.sum(csum_f.reshape(NW, 16, 16), axis=(0, 2))[:N_BINS]
    csum = (csum3 + 0.5 * count) * (1.0 / 2097151.0)
    safe = jnp.maximum(count, 1.0)
    gap = jnp.abs(csum / safe - asum / safe) * (count / N_ROWS)
    ece = jnp.sum(jnp.where(count > 0, gap, 0.0))
    return ece.reshape(1).astype(jnp.float32)


# speedup vs baseline: 1.0521x; 1.0521x over previous
import functools

import jax
import jax.numpy as jnp
import numpy as np
from jax import lax
from jax.experimental import pallas as pl
from jax.experimental.pallas import tpu as pltpu
from jax.experimental.pallas import tpu_sc as plsc

N_ROWS = 1048576
N_CLS = 64
N_BINS = 15
BLK = 65536
GRID = N_ROWS // BLK
N_CHUNKS = 1
CGRID = GRID // N_CHUNKS
CROWS = N_ROWS // N_CHUNKS
NW = 32
PER_W = CROWS // NW

_BOUNDS = [float(v) for v in np.linspace(0.0, 1.0, N_BINS + 1)[1:N_BINS].astype(np.float32)]


SUB = BLK // 128


def _tc_body(xt_ref, lab_ref, word_ref):
    xt = xt_ref[...]
    m = jnp.max(xt, axis=0, keepdims=True)
    e = jnp.exp(xt)
    ones_w = jnp.full((1, N_CLS), 1.0, jnp.float32)
    s = jax.lax.dot_general(ones_w, e, (((1,), (0,)), ((), ())),
                            preferred_element_type=jnp.float32)
    eqm = jnp.where(xt == m, 1.0, 0.0)
    iota_w = lax.broadcasted_iota(jnp.int32, (1, N_CLS), 1).astype(jnp.float32)
    pred = jax.lax.dot_general(iota_w, eqm, (((1,), (0,)), ((), ())),
                               preferred_element_type=jnp.float32)

    conf = jnp.exp(m.reshape(SUB, 128)) / s.reshape(SUB, 128)
    pred2 = pred.reshape(SUB, 128)
    acci = (pred2 == lab_ref[0].astype(jnp.float32)).astype(jnp.int32)
    b = (conf > _BOUNDS[0]).astype(jnp.int32)
    for bk in _BOUNDS[1:]:
        b += (conf > bk).astype(jnp.int32)

    conf_q = (conf * 2097151.0).astype(jnp.int32)
    word_ref[0] = b * 67108864 + acci * 2097152 + conf_q


def _tc_stage(logits_t, labels3, chunk):
    word_sd = jax.ShapeDtypeStruct((CGRID, SUB, 128), jnp.int32)
    base = chunk * CGRID
    row_spec = pl.BlockSpec((1, SUB, 128), lambda i: (base + i, 0, 0))
    out_spec = pl.BlockSpec((1, SUB, 128), lambda i: (i, 0, 0))
    word = pl.pallas_call(
        _tc_body,
        grid=(CGRID,),
        in_specs=[
            pl.BlockSpec((N_CLS, BLK), lambda i: (0, base + i)),
            row_spec,
        ],
        out_specs=out_spec,
        out_shape=word_sd,
    )(logits_t, labels3)
    return word.reshape(CROWS)


def _sc_body(word_hbm, cout_hbm, zout_hbm,
             word_v, csum_a, z_a, csum_b, z_b, cf_h):
    wid = lax.axis_index("s") * 2 + lax.axis_index("c")
    base = wid * PER_W
    pltpu.sync_copy(word_hbm.at[pl.ds(base, PER_W)], word_v)

    zeros16i = jnp.zeros((16,), jnp.int32)
    for r in range(16):
        sl = pl.ds(r * 16, 16)
        csum_a[sl] = zeros16i
        z_a[sl] = zeros16i
        csum_b[sl] = zeros16i
        z_b[sl] = zeros16i

    lane = lax.broadcasted_iota(jnp.int32, (16,), 0)

    def step(i, carry):
        off = i * 32
        w0 = word_v[pl.ds(off, 16)]
        w1 = word_v[pl.ds(off + 16, 16)]
        idx0 = lax.shift_right_logical(w0, 22) + lane
        idx1 = lax.shift_right_logical(w1, 22) + lane
        plsc.addupdate_scatter(csum_a, [idx0], lax.bitwise_and(w0, 2097151))
        plsc.addupdate_scatter(z_a, [idx0],
                               lax.bitwise_or(lax.shift_right_logical(w0, 21),
                                              32768))
        plsc.addupdate_scatter(csum_b, [idx1], lax.bitwise_and(w1, 2097151))
        plsc.addupdate_scatter(z_b, [idx1],
                               lax.bitwise_or(lax.shift_right_logical(w1, 21),
                                              32768))
        return carry

    lax.fori_loop(0, PER_W // 32, step, 0, unroll=4)

    for r in range(16):
        sl = pl.ds(r * 16, 16)
        cf_h[sl] = (csum_a[sl].astype(jnp.float32) +
                    csum_b[sl].astype(jnp.float32))
        z_a[sl] = z_a[sl] + z_b[sl]

    obase = wid * 256
    pltpu.sync_copy(cf_h, cout_hbm.at[pl.ds(obase, 256)])
    pltpu.sync_copy(z_a, zout_hbm.at[pl.ds(obase, 256)])


def _sc_stage(word):
    mesh = plsc.VectorSubcoreMesh(core_axis_name="c", subcore_axis_name="s")
    kern = pl.kernel(
        _sc_body,
        out_type=(jax.ShapeDtypeStruct((NW * 256,), jnp.float32),
                  jax.ShapeDtypeStruct((NW * 256,), jnp.int32)),
        mesh=mesh,
        compiler_params=pltpu.CompilerParams(needs_layout_passes=False),
        scratch_types=[
            pltpu.VMEM((PER_W,), jnp.int32),
            pltpu.VMEM((256,), jnp.int32),
            pltpu.VMEM((256,), jnp.int32),
            pltpu.VMEM((256,), jnp.int32),
            pltpu.VMEM((256,), jnp.int32),
            pltpu.VMEM((256,), jnp.float32),
        ],
    )
    return kern(word)


def kernel(logits, labels):
    labels3 = labels.astype(jnp.int32).reshape(GRID, SUB, 128)
    logits_t = logits.T
    csum_f = jnp.zeros((NW * 256,), jnp.float32)
    za = jnp.zeros((NW * 256,), jnp.int32)
    for h in range(N_CHUNKS):
        word = _tc_stage(logits_t, labels3, h)
        csum_p, z_p = _sc_stage(word)
        csum_f = csum_f + csum_p
        za = za + z_p
    za3 = za.reshape(NW, 16, 16)
    denom = (32768 + 32 * jnp.arange(16, dtype=jnp.int32)).reshape(1, 16, 1)
    count_c = za3 // denom
    acc_c = za3 - count_c * denom
    count = jnp.sum(count_c, axis=(0, 2)).astype(jnp.float32)[:N_BINS]
    asum = jnp.sum(acc_c, axis=(0, 2)).astype(jnp.float32)[:N_BINS]
    csum3 = jnp.sum(csum_f.reshape(NW, 16, 16), axis=(0, 2))[:N_BINS]
    csum = (csum3 + 0.5 * count) * (1.0 / 2097151.0)
    safe = jnp.maximum(count, 1.0)
    gap = jnp.abs(csum / safe - asum / safe) * (count / N_ROWS)
    ece = jnp.sum(jnp.where(count > 0, gap, 0.0))
    return ece.reshape(1).astype(jnp.float32)
